# baseline (speedup 1.0000x reference)
import functools

import jax
import jax.numpy as jnp
import numpy as np
from jax import lax
from jax.experimental import pallas as pl
from jax.experimental.pallas import tpu as pltpu
from jax.experimental.pallas import tpu_sc as plsc

N = 10000
NPAD = 10240
C = 128
H = 8
DH = 16
K = 16
NG = 1000
GPAD = 1024
SCALE = DH ** -0.5

NC = 2
NS = 16
NW = NC * NS

_F32 = jnp.float32

def _compute_gidx():
    with jax.default_device(jax.local_devices(backend="cpu")[0]):
        return np.asarray(jax.random.permutation(jax.random.key(42), N)[:NG])


try:
    _GIDX_NP = _compute_gidx()
except Exception:
    _GIDX_NP = None


def _get_gidx():
    if _GIDX_NP is None:
        return jax.random.permutation(jax.random.key(42), N)[:NG]
    return jnp.asarray(_GIDX_NP)


@functools.lru_cache(maxsize=None)
def _mesh():
    return plsc.VectorSubcoreMesh(core_axis_name="c", subcore_axis_name="s",
                                  num_cores=NC, num_subcores=NS)


def _wid():
    return lax.axis_index("s") * NC + lax.axis_index("c")


BF = NPAD * K
CH = 256
CHP = CH // K
RELW = CH * 16
NHALF = NPAD // 2
BFH = NHALF * K
BPWH = BFH // NW
NCHUNKH = BPWH // CH


@functools.lru_cache(maxsize=None)
def _sc_gather_half_kernel(pts_base):
    @functools.partial(
        pl.kernel,
        mesh=_mesh(),
        compiler_params=pltpu.CompilerParams(needs_layout_passes=False),
        out_type=[
            jax.ShapeDtypeStruct((BFH, C), _F32),
            jax.ShapeDtypeStruct((BFH * 16,), _F32),
        ],
        scratch_types=[
            pltpu.VMEM((2, 1, CH), jnp.int32),
            pltpu.VMEM((2 * CH, C), _F32),
            pltpu.VMEM((2 * RELW,), _F32),
            pltpu.VMEM((NPAD,), _F32),
            pltpu.VMEM((NPAD,), _F32),
            pltpu.VMEM((NPAD,), _F32),
            pltpu.SemaphoreType.DMA,
            pltpu.SemaphoreType.DMA,
            pltpu.SemaphoreType.DMA,
            pltpu.SemaphoreType.DMA,
            pltpu.SemaphoreType.DMA,
            pltpu.SemaphoreType.DMA,
        ],
    )
    def body_fn(idx_hbm, xtab, px_hbm, py_hbm, pz_hbm,
                xout, relout,
                idx_v, xbuf, relbuf, px_v, py_v, pz_v,
                sg0, sg1, so0, so1, sr0, sr1):
        base = _wid() * BPWH
        base_pts = pts_base + _wid() * (BPWH // K)
        relbase = _wid() * BPWH * 16
        sg = (sg0, sg1)
        so = (so0, so1)
        sr = (sr0, sr1)

        pltpu.sync_copy(px_hbm, px_v)
        pltpu.sync_copy(py_hbm, py_v)
        pltpu.sync_copy(pz_hbm, pz_v)

        zeros16 = jnp.zeros((16,), _F32)

        def zbody(j, _):
            relbuf[pl.ds(j * 16, 16)] = zeros16
            return 0

        lax.fori_loop(0, 2 * RELW // 16, zbody, 0)

        pltpu.sync_copy(idx_hbm.at[pl.ds(pl.multiple_of(base, 8), CH)],
                        idx_v.at[0, 0])
        pltpu.async_copy(xtab.at[idx_v.at[0, 0]], xbuf.at[pl.ds(0, CH)], sg0)

        def pair(g, _):
            for b in range(2):
                it = g * 2 + b
                nb = 1 - b
                @pl.when(it + 1 < NCHUNKH)
                def _():
                    @pl.when(it >= 1)
                    def _():
                        pltpu.make_async_copy(
                            xbuf.at[pl.ds(nb * CH, CH)],
                            xout.at[pl.ds(0, CH)], so[nb]).wait()
                        pltpu.make_async_copy(
                            relbuf.at[pl.ds(nb * RELW, RELW)],
                            relout.at[pl.ds(0, RELW)], sr[nb]).wait()
                    off1 = pl.multiple_of(base + (it + 1) * CH, 8)
                    pltpu.sync_copy(idx_hbm.at[pl.ds(off1, CH)],
                                    idx_v.at[nb, 0])
                    pltpu.async_copy(xtab.at[idx_v.at[nb, 0]],
                                     xbuf.at[pl.ds(nb * CH, CH)], sg[nb])

                lane = lax.iota(jnp.int32, 16)
                for i in range(CHP):
                    nbr = idx_v[b, 0, pl.ds(i * K, K)]
                    pid = jnp.full((16,), base_pts + it * CHP + i,
                                   jnp.int32)
                    lidx = lane * 16 + i * (K * 16) + b * RELW
                    relx = (plsc.load_gather(px_v, [nbr])
                            - plsc.load_gather(px_v, [pid]))
                    rely = (plsc.load_gather(py_v, [nbr])
                            - plsc.load_gather(py_v, [pid]))
                    relz = (plsc.load_gather(pz_v, [nbr])
                            - plsc.load_gather(pz_v, [pid]))
                    plsc.store_scatter(relbuf, [lidx], relx)
                    plsc.store_scatter(relbuf, [lidx + 1], rely)
                    plsc.store_scatter(relbuf, [lidx + 2], relz)

                off = pl.multiple_of(base + it * CH, 8)
                pltpu.make_async_copy(
                    xtab.at[pl.ds(0, CH)], xbuf.at[pl.ds(b * CH, CH)],
                    sg[b]).wait()
                pltpu.async_copy(xbuf.at[pl.ds(b * CH, CH)],
                                 xout.at[pl.ds(off, CH)], so[b])
                pltpu.async_copy(relbuf.at[pl.ds(b * RELW, RELW)],
                                 relout.at[pl.ds(relbase + it * RELW, RELW)],
                                 sr[b])
            return 0

        lax.fori_loop(0, NCHUNKH // 2, pair, 0)

        for b in range(2):
            pltpu.make_async_copy(xbuf.at[pl.ds(b * CH, CH)],
                                  xout.at[pl.ds(0, CH)], so[b]).wait()
            pltpu.make_async_copy(relbuf.at[pl.ds(b * RELW, RELW)],
                                  relout.at[pl.ds(0, RELW)], sr[b]).wait()

    return body_fn


def _sc_gather_half(pts_base, idx_half, x, px, py, pz):
    return _sc_gather_half_kernel(pts_base)(idx_half, x, px, py, pz)


GW = GPAD // NW


@functools.lru_cache(maxsize=None)
def _sc_gather_anchors_kernel():
    @functools.partial(
        pl.kernel,
        mesh=_mesh(),
        out_type=[
            jax.ShapeDtypeStruct((GPAD, C), _F32),
            jax.ShapeDtypeStruct((GPAD, C), _F32),
        ],
        scratch_types=[
            pltpu.VMEM((GW,), jnp.int32),
            pltpu.VMEM((GW, C), _F32),
            pltpu.VMEM((GW, C), _F32),
            pltpu.SemaphoreType.DMA,
            pltpu.SemaphoreType.DMA,
        ],
    )
    def body_fn(gidx_hbm, xtab, ptab, xout, pout,
                idx_v, xbuf, pbuf, sx, sp):
        off = pl.multiple_of(_wid() * GW, 8)
        pltpu.sync_copy(gidx_hbm.at[pl.ds(off, GW)], idx_v)
        cx = pltpu.async_copy(xtab.at[idx_v], xbuf, sx)
        cp = pltpu.async_copy(ptab.at[idx_v], pbuf, sp)
        cx.wait()
        cp.wait()
        pltpu.sync_copy(xbuf, xout.at[pl.ds(off, GW)])
        pltpu.sync_copy(pbuf, pout.at[pl.ds(off, GW)])

    return body_fn


def _sc_gather_anchors(gidx, xl, p16):
    return _sc_gather_anchors_kernel()(gidx, xl, p16)


RW = NPAD // NW


@functools.lru_cache(maxsize=None)
def _sc_gather_rows_kernel():
    @functools.partial(
        pl.kernel,
        mesh=_mesh(),
        out_type=jax.ShapeDtypeStruct((NPAD, C), _F32),
        scratch_types=[
            pltpu.VMEM((RW,), jnp.int32),
            pltpu.VMEM((RW, C), _F32),
            pltpu.SemaphoreType.DMA,
        ],
    )
    def body_fn(nidx_hbm, xtab, xout, idx_v, xbuf, sx):
        off = pl.multiple_of(_wid() * RW, 8)
        pltpu.sync_copy(nidx_hbm.at[pl.ds(off, RW)], idx_v)
        pltpu.async_copy(xtab.at[idx_v], xbuf, sx).wait()
        pltpu.sync_copy(xbuf, xout.at[pl.ds(off, RW)])

    return body_fn


def _sc_gather_rows(nidx, xg):
    return _sc_gather_rows_kernel()(nidx, xg)


def _gelu(x):
    return 0.5 * x * (1.0 + lax.erf(x * (2.0 ** -0.5)))


def _ln(x, g, b):
    m = jnp.mean(x, axis=-1, keepdims=True)
    v = jnp.mean((x - m) ** 2, axis=-1, keepdims=True)
    return (x - m) / jnp.sqrt(v + 1e-5) * g + b


def _row(ref):
    return ref[0:1, :]


P_LOC = 128


def _head_sum_mat():
    cid = lax.broadcasted_iota(jnp.int32, (C, H), 0) // DH
    hid = lax.broadcasted_iota(jnp.int32, (C, H), 1)
    return jnp.where(cid == hid, 1.0, 0.0).astype(_F32)


def _local_attn_body(x_ref, xf_ref, relp_ref,
                     wq_ref, bq_ref, wk_ref, bk_ref, wv_ref, bv_ref,
                     bd1_ref, b1_ref, bd2_ref, b2_ref, wo_ref, bo_ref,
                     g_ref, b_ref, o_ref):
    P = P_LOC
    x = x_ref[...]
    xf = xf_ref[...]
    relp = relp_ref[...]

    q = jnp.dot(x, wq_ref[...], preferred_element_type=_F32) + _row(bq_ref)
    kf = jnp.dot(xf, wk_ref[...], preferred_element_type=_F32) + _row(bk_ref)
    vf = jnp.dot(xf, wv_ref[...], preferred_element_type=_F32) + _row(bv_ref)

    E = _head_sum_mat()
    prod = (kf.reshape(P, K, C) * q[:, None, :]).reshape(P * K, C)
    s = jnp.dot(prod, E, preferred_element_type=_F32) * SCALE

    h1 = jnp.dot(relp, bd1_ref[...], preferred_element_type=_F32) + _row(b1_ref)
    h1 = _gelu(h1)
    biasp = jnp.dot(h1, bd2_ref[...], preferred_element_type=_F32) + _row(b2_ref)
    PP = P * K * 16 // C
    brows = jnp.broadcast_to(biasp[:, None, :], (PP, 8, 64)).reshape(P * K, 64)
    rowmod = lax.broadcasted_iota(jnp.int32, (P * K, 64), 0) % 8
    colj = lax.broadcasted_iota(jnp.int32, (P * K, 64), 1) // 8
    masked = jnp.where(rowmod == colj, brows, 0.0)
    hsum = jnp.zeros((64, H), _F32)
    hid8 = lax.broadcasted_iota(jnp.int32, (64, H), 0) % 8
    hcol = lax.broadcasted_iota(jnp.int32, (64, H), 1)
    hsum = jnp.where(hid8 == hcol, 1.0, 0.0).astype(_F32)
    bias = jnp.dot(masked, hsum, preferred_element_type=_F32)

    a = (s + bias).reshape(P, K, H)
    m = jnp.max(a, axis=1, keepdims=True)
    e = jnp.exp(a - m)
    den = jnp.sum(e, axis=1, keepdims=True)
    p = (e / den).reshape(P * K, H)

    pbc = jnp.dot(p, E.T, preferred_element_type=_F32)
    out = (pbc * vf).reshape(P, K, C).sum(axis=1)
    out = jnp.dot(out, wo_ref[...], preferred_element_type=_F32) + _row(bo_ref)
    o_ref[...] = _ln(x + out, _row(g_ref), _row(b_ref))


def _tc_local_attn(xp, xf, relp, wqT, bq, wkT, bk, wvT, bv,
                   bd1, b1, bd2, b2, woT, bo, g, b):
    npts = xp.shape[0]
    P = P_LOC
    PP = P * K * 16 // C
    grid = npts // P
    rblk = pl.BlockSpec((P, C), lambda i: (i, 0))
    nblk = pl.BlockSpec((P * K, C), lambda i: (i, 0))
    relblk = pl.BlockSpec((PP, C), lambda i: (i, 0))
    bd1s = pl.BlockSpec((C, 512), lambda i: (0, 0))
    b1s = pl.BlockSpec((8, 512), lambda i: (0, 0))
    bd2s = pl.BlockSpec((512, 64), lambda i: (0, 0))
    b2s = pl.BlockSpec((8, 64), lambda i: (0, 0))
    full = pl.BlockSpec((C, C), lambda i: (0, 0))
    brow = pl.BlockSpec((8, C), lambda i: (0, 0))
    return pl.pallas_call(
        _local_attn_body,
        grid=(grid,),
        in_specs=[rblk, nblk, relblk,
                  full, brow, full, brow, full, brow,
                  bd1s, b1s, bd2s, b2s, full, brow, brow, brow],
        out_specs=rblk,
        out_shape=jax.ShapeDtypeStruct((npts, C), _F32),
    )(xp, xf, relp, wqT, bq, wkT, bk, wvT, bv,
      bd1, b1, bd2, b2, woT, bo, g, b)


def _qkv_body(x_ref, w_ref, b_ref, o_ref):
    o_ref[...] = (jnp.dot(x_ref[...], w_ref[...], preferred_element_type=_F32)
                  + _row(b_ref))


def _tc_qkv(xg, wT, b):
    return pl.pallas_call(
        _qkv_body,
        grid=(1,),
        in_specs=[pl.BlockSpec((GPAD, C), lambda i: (0, 0)),
                  pl.BlockSpec((C, 3 * C), lambda i: (0, 0)),
                  pl.BlockSpec((8, 3 * C), lambda i: (0, 0))],
        out_specs=pl.BlockSpec((GPAD, 3 * C), lambda i: (0, 0)),
        out_shape=jax.ShapeDtypeStruct((GPAD, 3 * C), _F32),
    )(xg, wT, b)


QB = 256


def _global_body(xg_ref, q_ref, k_ref, v_ref, wgo_ref, bgo_ref,
                 g1g_ref, g1b_ref, wf1_ref, bf1_ref, wf2_ref, bf2_ref,
                 g2g_ref, g2b_ref, o_ref):
    q = q_ref[...]
    k = k_ref[...]
    v = v_ref[...]
    colid = lax.broadcasted_iota(jnp.int32, (QB, GPAD), 1)
    outs = []
    for h in range(H):
        qh = q[:, h * DH:(h + 1) * DH]
        kh = k[:, h * DH:(h + 1) * DH]
        vh = v[:, h * DH:(h + 1) * DH]
        s = lax.dot_general(qh, kh, (((1,), (1,)), ((), ())),
                            preferred_element_type=_F32) * SCALE
        s = jnp.where(colid < NG, s, -1e30)
        m = jnp.max(s, axis=-1, keepdims=True)
        e = jnp.exp(s - m)
        p = e / jnp.sum(e, axis=-1, keepdims=True)
        outs.append(jnp.dot(p, vh, preferred_element_type=_F32))
    og = jnp.concatenate(outs, axis=-1)
    og = jnp.dot(og, wgo_ref[...], preferred_element_type=_F32) + _row(bgo_ref)
    x1 = _ln(xg_ref[...] + og, _row(g1g_ref), _row(g1b_ref))
    hid = jnp.dot(x1, wf1_ref[...], preferred_element_type=_F32) + _row(bf1_ref)
    hid = _gelu(hid)
    ff = jnp.dot(hid, wf2_ref[...], preferred_element_type=_F32) + _row(bf2_ref)
    o_ref[...] = _ln(x1 + ff, _row(g2g_ref), _row(g2b_ref))


def _tc_global(xg, qg, kg, vg, wgoT, bgo, g1g, g1b, wf1T, bf1, wf2T, bf2,
               g2g, g2b):
    grid = GPAD // QB
    qblk = pl.BlockSpec((QB, C), lambda i: (i, 0))
    kfull = pl.BlockSpec((GPAD, C), lambda i: (0, 0))
    full = pl.BlockSpec((C, C), lambda i: (0, 0))
    brow = pl.BlockSpec((8, C), lambda i: (0, 0))
    w1s = pl.BlockSpec((C, 4 * C), lambda i: (0, 0))
    b1s = pl.BlockSpec((8, 4 * C), lambda i: (0, 0))
    w2s = pl.BlockSpec((4 * C, C), lambda i: (0, 0))
    return pl.pallas_call(
        _global_body,
        grid=(grid,),
        in_specs=[qblk, qblk, kfull, kfull, full, brow, brow, brow,
                  w1s, b1s, w2s, brow, brow, brow],
        out_specs=qblk,
        out_shape=jax.ShapeDtypeStruct((GPAD, C), _F32),
    )(xg, qg, kg, vg, wgoT, bgo, g1g, g1b, wf1T, bf1, wf2T, bf2, g2g, g2b)


NB_BLK = 1024


def _nearest_body(pos_ref, pgT_ref, o_ref):
    pos = pos_ref[...]
    pgT = pgT_ref[...]
    colid = lax.broadcasted_iota(jnp.int32, (NB_BLK, GPAD), 1)
    d2 = jnp.zeros((NB_BLK, GPAD), _F32)
    for cdim in range(3):
        dc = pos[:, cdim:cdim + 1] - pgT[cdim:cdim + 1, :]
        d2 = d2 + dc * dc
    d = jnp.sqrt(d2)
    d = jnp.where(colid < NG, d, jnp.inf)
    dmin = jnp.min(d, axis=-1, keepdims=True)
    near = jnp.min(jnp.where(d == dmin, colid, jnp.int32(2**30)), axis=-1)
    o_ref[...] = near.astype(jnp.int32).reshape(NB_BLK // C, C)


def _tc_nearest(pos16, posgT):
    grid = NPAD // NB_BLK
    return pl.pallas_call(
        _nearest_body,
        grid=(grid,),
        in_specs=[pl.BlockSpec((NB_BLK, C), lambda i: (i, 0)),
                  pl.BlockSpec((C, GPAD), lambda i: (0, 0))],
        out_specs=pl.BlockSpec((NB_BLK // C, C), lambda i: (i, 0)),
        out_shape=jax.ShapeDtypeStruct((NPAD // C, C), jnp.int32),
    )(pos16, posgT)


def _fuse_body(xl_ref, xgf_ref, wa_ref, wb_ref, bf_ref, g_ref, b_ref, o_ref):
    f = (jnp.dot(xl_ref[...], wa_ref[...], preferred_element_type=_F32)
         + jnp.dot(xgf_ref[...], wb_ref[...], preferred_element_type=_F32)
         + _row(bf_ref))
    f = _ln(f, _row(g_ref), _row(b_ref))
    o_ref[...] = _gelu(f)


def _tc_fuse(xl, xgf, waT, wbT, bf, g, b):
    blk = 1024
    grid = NPAD // blk
    rblk = pl.BlockSpec((blk, C), lambda i: (i, 0))
    full = pl.BlockSpec((C, C), lambda i: (0, 0))
    brow = pl.BlockSpec((8, C), lambda i: (0, 0))
    return pl.pallas_call(
        _fuse_body,
        grid=(grid,),
        in_specs=[rblk, rblk, full, full, brow, brow, brow],
        out_specs=rblk,
        out_shape=jax.ShapeDtypeStruct((NPAD, C), _F32),
    )(xl, xgf, waT, wbT, bf, g, b)


def _tile8(v):
    return jnp.tile(v[None, :], (8, 1))


def kernel(x, pos, idx, params):
    p = params
    xp = jnp.zeros((NPAD, C), _F32).at[:N].set(x)
    pos16 = jnp.zeros((NPAD, C), _F32).at[:N, :3].set(pos)
    idxp = jnp.zeros((NPAD, K), jnp.int32).at[:N].set(idx)
    idxflat = idxp.reshape(BF)

    wqT, bq = p['q'][0].T, _tile8(p['q'][1])
    wkT, bk = p['k'][0].T, _tile8(p['k'][1])
    wvT, bv = p['v'][0].T, _tile8(p['v'][1])
    woT, bo = p['o'][0].T, _tile8(p['o'][1])
    w1p = jnp.zeros((16, 64), _F32).at[:3].set(p['pe1'][0].T)
    bd1 = jnp.kron(jnp.eye(8, dtype=_F32), w1p)
    b1 = jnp.tile(p['pe1'][1], 8)[None, :].repeat(8, 0)
    bd2 = jnp.kron(jnp.eye(8, dtype=_F32), p['pe2'][0].T)
    b2 = jnp.tile(p['pe2'][1], 8)[None, :].repeat(8, 0)
    lng, lnb = _tile8(p['ln_local'][0]), _tile8(p['ln_local'][1])
    wqkvT, bqkv = p['qkv'][0].T, _tile8(p['qkv'][1])
    wgoT, bgo = p['go'][0].T, _tile8(p['go'][1])
    g1g, g1b = _tile8(p['gn1'][0]), _tile8(p['gn1'][1])
    g2g, g2b = _tile8(p['gn2'][0]), _tile8(p['gn2'][1])
    wf1T, bf1 = p['ffn1'][0].T, _tile8(p['ffn1'][1])
    wf2T, bf2 = p['ffn2'][0].T, _tile8(p['ffn2'][1])
    waT = p['fuse'][0][:, :C].T
    wbT = p['fuse'][0][:, C:].T
    bfu = _tile8(p['fuse'][1])
    flg, flb = _tile8(p['fuse_ln'][0]), _tile8(p['fuse_ln'][1])

    gidx = jnp.zeros((GPAD,), jnp.int32).at[:NG].set(
        jnp.asarray(_get_gidx(), jnp.int32))

    px = jnp.zeros((NPAD,), _F32).at[:N].set(pos[:, 0])
    py = jnp.zeros((NPAD,), _F32).at[:N].set(pos[:, 1])
    pz = jnp.zeros((NPAD,), _F32).at[:N].set(pos[:, 2])

    xls = []
    for h in range(2):
        idx_h = lax.slice_in_dim(idxflat, h * BFH, (h + 1) * BFH)
        xf, relflat = _sc_gather_half(h * NHALF, idx_h, xp, px, py, pz)
        relp = relflat.reshape(BFH * 16 // C, C)
        xp_h = lax.slice_in_dim(xp, h * NHALF, (h + 1) * NHALF)
        xls.append(_tc_local_attn(xp_h, xf, relp, wqT, bq, wkT, bk, wvT, bv,
                                  bd1, b1, bd2, b2, woT, bo, lng, lnb))
    x_local = jnp.concatenate(xls, axis=0)

    xg_in, posg = _sc_gather_anchors(gidx, x_local, pos16)
    qkv = _tc_qkv(xg_in, wqkvT, bqkv)
    qg, kg, vg = qkv[:, :C], qkv[:, C:2 * C], qkv[:, 2 * C:]
    xg = _tc_global(xg_in, qg, kg, vg, wgoT, bgo, g1g, g1b,
                    wf1T, bf1, wf2T, bf2, g2g, g2b)

    nearest = _tc_nearest(pos16, posg.T).reshape(NPAD)
    xgf = _sc_gather_rows(nearest, xg)

    out = _tc_fuse(x_local, xgf, waT, wbT, bfu, flg, flb)
    return out[:N]


# speedup vs baseline: 1.0232x; 1.0232x over previous
import functools

import jax
import jax.numpy as jnp
import numpy as np
from jax import lax
from jax.experimental import pallas as pl
from jax.experimental.pallas import tpu as pltpu
from jax.experimental.pallas import tpu_sc as plsc

N = 10000
NPAD = 10240
C = 128
H = 8
DH = 16
K = 16
NG = 1000
GPAD = 1024
SCALE = DH ** -0.5

NC = 2
NS = 16
NW = NC * NS

_F32 = jnp.float32

def _compute_gidx():
    with jax.default_device(jax.local_devices(backend="cpu")[0]):
        return np.asarray(jax.random.permutation(jax.random.key(42), N)[:NG])


try:
    _GIDX_NP = _compute_gidx()
except Exception:
    _GIDX_NP = None


def _get_gidx():
    if _GIDX_NP is None:
        return jax.random.permutation(jax.random.key(42), N)[:NG]
    return jnp.asarray(_GIDX_NP)


@functools.lru_cache(maxsize=None)
def _mesh():
    return plsc.VectorSubcoreMesh(core_axis_name="c", subcore_axis_name="s",
                                  num_cores=NC, num_subcores=NS)


def _wid():
    return lax.axis_index("s") * NC + lax.axis_index("c")


BF = NPAD * K
CH = 256
CHP = CH // K
RELW = CH * 16
NHALF = NPAD // 2
BFH = NHALF * K
BPWH = BFH // NW
NCHUNKH = BPWH // CH


@functools.lru_cache(maxsize=None)
def _sc_gather_half_kernel(pts_base):
    @functools.partial(
        pl.kernel,
        mesh=_mesh(),
        compiler_params=pltpu.CompilerParams(needs_layout_passes=False),
        out_type=[
            jax.ShapeDtypeStruct((BFH, C), _F32),
            jax.ShapeDtypeStruct((BFH * 16,), _F32),
        ],
        scratch_types=[
            pltpu.VMEM((2, 1, CH), jnp.int32),
            pltpu.VMEM((2 * CH, C), _F32),
            pltpu.VMEM((2 * RELW,), _F32),
            pltpu.VMEM((NPAD,), _F32),
            pltpu.VMEM((NPAD,), _F32),
            pltpu.VMEM((NPAD,), _F32),
            pltpu.SemaphoreType.DMA,
            pltpu.SemaphoreType.DMA,
            pltpu.SemaphoreType.DMA,
            pltpu.SemaphoreType.DMA,
            pltpu.SemaphoreType.DMA,
            pltpu.SemaphoreType.DMA,
        ],
    )
    def body_fn(idx_hbm, xtab, px_hbm, py_hbm, pz_hbm,
                xout, relout,
                idx_v, xbuf, relbuf, px_v, py_v, pz_v,
                sg0, sg1, so0, so1, sr0, sr1):
        base = _wid() * BPWH
        base_pts = pts_base + _wid() * (BPWH // K)
        relbase = _wid() * BPWH * 16
        sg = (sg0, sg1)
        so = (so0, so1)
        sr = (sr0, sr1)

        pltpu.sync_copy(px_hbm, px_v)
        pltpu.sync_copy(py_hbm, py_v)
        pltpu.sync_copy(pz_hbm, pz_v)

        zeros16 = jnp.zeros((16,), _F32)

        def zbody(j, _):
            relbuf[pl.ds(j * 16, 16)] = zeros16
            return 0

        lax.fori_loop(0, 2 * RELW // 16, zbody, 0)

        pltpu.sync_copy(idx_hbm.at[pl.ds(pl.multiple_of(base, 8), CH)],
                        idx_v.at[0, 0])
        pltpu.async_copy(xtab.at[idx_v.at[0, 0]], xbuf.at[pl.ds(0, CH)], sg0)

        def pair(g, _):
            for b in range(2):
                it = g * 2 + b
                nb = 1 - b
                @pl.when(it + 1 < NCHUNKH)
                def _():
                    @pl.when(it >= 1)
                    def _():
                        pltpu.make_async_copy(
                            xbuf.at[pl.ds(nb * CH, CH)],
                            xout.at[pl.ds(0, CH)], so[nb]).wait()
                        pltpu.make_async_copy(
                            relbuf.at[pl.ds(nb * RELW, RELW)],
                            relout.at[pl.ds(0, RELW)], sr[nb]).wait()
                    off1 = pl.multiple_of(base + (it + 1) * CH, 8)
                    pltpu.sync_copy(idx_hbm.at[pl.ds(off1, CH)],
                                    idx_v.at[nb, 0])
                    pltpu.async_copy(xtab.at[idx_v.at[nb, 0]],
                                     xbuf.at[pl.ds(nb * CH, CH)], sg[nb])

                lane = lax.iota(jnp.int32, 16)
                for i in range(CHP):
                    nbr = idx_v[b, 0, pl.ds(i * K, K)]
                    pid = jnp.full((16,), base_pts + it * CHP + i,
                                   jnp.int32)
                    lidx = lane * 16 + i * (K * 16) + b * RELW
                    relx = (plsc.load_gather(px_v, [nbr])
                            - plsc.load_gather(px_v, [pid]))
                    rely = (plsc.load_gather(py_v, [nbr])
                            - plsc.load_gather(py_v, [pid]))
                    relz = (plsc.load_gather(pz_v, [nbr])
                            - plsc.load_gather(pz_v, [pid]))
                    plsc.store_scatter(relbuf, [lidx], relx)
                    plsc.store_scatter(relbuf, [lidx + 1], rely)
                    plsc.store_scatter(relbuf, [lidx + 2], relz)

                off = pl.multiple_of(base + it * CH, 8)
                pltpu.make_async_copy(
                    xtab.at[pl.ds(0, CH)], xbuf.at[pl.ds(b * CH, CH)],
                    sg[b]).wait()
                pltpu.async_copy(xbuf.at[pl.ds(b * CH, CH)],
                                 xout.at[pl.ds(off, CH)], so[b])
                pltpu.async_copy(relbuf.at[pl.ds(b * RELW, RELW)],
                                 relout.at[pl.ds(relbase + it * RELW, RELW)],
                                 sr[b])
            return 0

        lax.fori_loop(0, NCHUNKH // 2, pair, 0)

        for b in range(2):
            pltpu.make_async_copy(xbuf.at[pl.ds(b * CH, CH)],
                                  xout.at[pl.ds(0, CH)], so[b]).wait()
            pltpu.make_async_copy(relbuf.at[pl.ds(b * RELW, RELW)],
                                  relout.at[pl.ds(0, RELW)], sr[b]).wait()

    return body_fn


def _sc_gather_half(pts_base, idx_half, x, px, py, pz):
    return _sc_gather_half_kernel(pts_base)(idx_half, x, px, py, pz)


GW = GPAD // NW


@functools.lru_cache(maxsize=None)
def _sc_gather_anchors_kernel():
    @functools.partial(
        pl.kernel,
        mesh=_mesh(),
        out_type=[
            jax.ShapeDtypeStruct((GPAD, C), _F32),
            jax.ShapeDtypeStruct((GPAD, C), _F32),
        ],
        scratch_types=[
            pltpu.VMEM((GW,), jnp.int32),
            pltpu.VMEM((GW, C), _F32),
            pltpu.VMEM((GW, C), _F32),
            pltpu.SemaphoreType.DMA,
            pltpu.SemaphoreType.DMA,
        ],
    )
    def body_fn(gidx_hbm, xtab, ptab, xout, pout,
                idx_v, xbuf, pbuf, sx, sp):
        off = pl.multiple_of(_wid() * GW, 8)
        pltpu.sync_copy(gidx_hbm.at[pl.ds(off, GW)], idx_v)
        cx = pltpu.async_copy(xtab.at[idx_v], xbuf, sx)
        cp = pltpu.async_copy(ptab.at[idx_v], pbuf, sp)
        cx.wait()
        cp.wait()
        pltpu.sync_copy(xbuf, xout.at[pl.ds(off, GW)])
        pltpu.sync_copy(pbuf, pout.at[pl.ds(off, GW)])

    return body_fn


def _sc_gather_anchors(gidx, xl, p16):
    return _sc_gather_anchors_kernel()(gidx, xl, p16)


RW = NPAD // NW


@functools.lru_cache(maxsize=None)
def _sc_gather_rows_kernel():
    @functools.partial(
        pl.kernel,
        mesh=_mesh(),
        out_type=jax.ShapeDtypeStruct((NPAD, C), _F32),
        scratch_types=[
            pltpu.VMEM((RW,), jnp.int32),
            pltpu.VMEM((RW, C), _F32),
            pltpu.SemaphoreType.DMA,
        ],
    )
    def body_fn(nidx_hbm, xtab, xout, idx_v, xbuf, sx):
        off = pl.multiple_of(_wid() * RW, 8)
        pltpu.sync_copy(nidx_hbm.at[pl.ds(off, RW)], idx_v)
        pltpu.async_copy(xtab.at[idx_v], xbuf, sx).wait()
        pltpu.sync_copy(xbuf, xout.at[pl.ds(off, RW)])

    return body_fn


def _sc_gather_rows(nidx, xg):
    return _sc_gather_rows_kernel()(nidx, xg)


def _gelu(x):
    return 0.5 * x * (1.0 + lax.erf(x * (2.0 ** -0.5)))


def _ln(x, g, b):
    m = jnp.mean(x, axis=-1, keepdims=True)
    v = jnp.mean((x - m) ** 2, axis=-1, keepdims=True)
    return (x - m) / jnp.sqrt(v + 1e-5) * g + b


def _row(ref):
    return ref[0:1, :]


P_LOC = 256


def _head_sum_mat():
    cid = lax.broadcasted_iota(jnp.int32, (C, H), 0) // DH
    hid = lax.broadcasted_iota(jnp.int32, (C, H), 1)
    return jnp.where(cid == hid, 1.0, 0.0).astype(_F32)


def _local_attn_body(x_ref, xf_ref, relp_ref,
                     wq_ref, bq_ref, wk_ref, bk_ref, wv_ref, bv_ref,
                     bd1_ref, b1_ref, bd2_ref, b2_ref, wo_ref, bo_ref,
                     g_ref, b_ref, o_ref):
    P = P_LOC
    x = x_ref[...]
    xf = xf_ref[...]
    relp = relp_ref[...]

    q = jnp.dot(x, wq_ref[...], preferred_element_type=_F32) + _row(bq_ref)
    kf = jnp.dot(xf, wk_ref[...], preferred_element_type=_F32) + _row(bk_ref)
    vf = jnp.dot(xf, wv_ref[...], preferred_element_type=_F32) + _row(bv_ref)

    E = _head_sum_mat()
    prod = (kf.reshape(P, K, C) * q[:, None, :]).reshape(P * K, C)
    s = jnp.dot(prod, E, preferred_element_type=_F32) * SCALE

    h1 = jnp.dot(relp, bd1_ref[...], preferred_element_type=_F32) + _row(b1_ref)
    h1 = _gelu(h1)
    biasp = jnp.dot(h1, bd2_ref[...], preferred_element_type=_F32) + _row(b2_ref)
    PP = P * K * 16 // C
    brows = jnp.broadcast_to(biasp[:, None, :], (PP, 8, 64)).reshape(P * K, 64)
    rowmod = lax.broadcasted_iota(jnp.int32, (P * K, 64), 0) % 8
    colj = lax.broadcasted_iota(jnp.int32, (P * K, 64), 1) // 8
    masked = jnp.where(rowmod == colj, brows, 0.0)
    hsum = jnp.zeros((64, H), _F32)
    hid8 = lax.broadcasted_iota(jnp.int32, (64, H), 0) % 8
    hcol = lax.broadcasted_iota(jnp.int32, (64, H), 1)
    hsum = jnp.where(hid8 == hcol, 1.0, 0.0).astype(_F32)
    bias = jnp.dot(masked, hsum, preferred_element_type=_F32)

    a = (s + bias).reshape(P, K, H)
    m = jnp.max(a, axis=1, keepdims=True)
    e = jnp.exp(a - m)
    den = jnp.sum(e, axis=1, keepdims=True)
    p = (e / den).reshape(P * K, H)

    pbc = jnp.dot(p, E.T, preferred_element_type=_F32)
    out = (pbc * vf).reshape(P, K, C).sum(axis=1)
    out = jnp.dot(out, wo_ref[...], preferred_element_type=_F32) + _row(bo_ref)
    o_ref[...] = _ln(x + out, _row(g_ref), _row(b_ref))


def _tc_local_attn(xp, xf, relp, wqT, bq, wkT, bk, wvT, bv,
                   bd1, b1, bd2, b2, woT, bo, g, b):
    npts = xp.shape[0]
    P = P_LOC
    PP = P * K * 16 // C
    grid = npts // P
    rblk = pl.BlockSpec((P, C), lambda i: (i, 0))
    nblk = pl.BlockSpec((P * K, C), lambda i: (i, 0))
    relblk = pl.BlockSpec((PP, C), lambda i: (i, 0))
    bd1s = pl.BlockSpec((C, 512), lambda i: (0, 0))
    b1s = pl.BlockSpec((8, 512), lambda i: (0, 0))
    bd2s = pl.BlockSpec((512, 64), lambda i: (0, 0))
    b2s = pl.BlockSpec((8, 64), lambda i: (0, 0))
    full = pl.BlockSpec((C, C), lambda i: (0, 0))
    brow = pl.BlockSpec((8, C), lambda i: (0, 0))
    return pl.pallas_call(
        _local_attn_body,
        grid=(grid,),
        in_specs=[rblk, nblk, relblk,
                  full, brow, full, brow, full, brow,
                  bd1s, b1s, bd2s, b2s, full, brow, brow, brow],
        out_specs=rblk,
        out_shape=jax.ShapeDtypeStruct((npts, C), _F32),
    )(xp, xf, relp, wqT, bq, wkT, bk, wvT, bv,
      bd1, b1, bd2, b2, woT, bo, g, b)


def _qkv_body(x_ref, w_ref, b_ref, o_ref):
    o_ref[...] = (jnp.dot(x_ref[...], w_ref[...], preferred_element_type=_F32)
                  + _row(b_ref))


def _tc_qkv(xg, wT, b):
    return pl.pallas_call(
        _qkv_body,
        grid=(1,),
        in_specs=[pl.BlockSpec((GPAD, C), lambda i: (0, 0)),
                  pl.BlockSpec((C, 3 * C), lambda i: (0, 0)),
                  pl.BlockSpec((8, 3 * C), lambda i: (0, 0))],
        out_specs=pl.BlockSpec((GPAD, 3 * C), lambda i: (0, 0)),
        out_shape=jax.ShapeDtypeStruct((GPAD, 3 * C), _F32),
    )(xg, wT, b)


QB = 256


def _global_body(xg_ref, q_ref, k_ref, v_ref, wgo_ref, bgo_ref,
                 g1g_ref, g1b_ref, wf1_ref, bf1_ref, wf2_ref, bf2_ref,
                 g2g_ref, g2b_ref, o_ref):
    q = q_ref[...]
    k = k_ref[...]
    v = v_ref[...]
    colid = lax.broadcasted_iota(jnp.int32, (QB, GPAD), 1)
    outs = []
    for h in range(H):
        qh = q[:, h * DH:(h + 1) * DH]
        kh = k[:, h * DH:(h + 1) * DH]
        vh = v[:, h * DH:(h + 1) * DH]
        s = lax.dot_general(qh, kh, (((1,), (1,)), ((), ())),
                            preferred_element_type=_F32) * SCALE
        s = jnp.where(colid < NG, s, -1e30)
        m = jnp.max(s, axis=-1, keepdims=True)
        e = jnp.exp(s - m)
        p = e / jnp.sum(e, axis=-1, keepdims=True)
        outs.append(jnp.dot(p, vh, preferred_element_type=_F32))
    og = jnp.concatenate(outs, axis=-1)
    og = jnp.dot(og, wgo_ref[...], preferred_element_type=_F32) + _row(bgo_ref)
    x1 = _ln(xg_ref[...] + og, _row(g1g_ref), _row(g1b_ref))
    hid = jnp.dot(x1, wf1_ref[...], preferred_element_type=_F32) + _row(bf1_ref)
    hid = _gelu(hid)
    ff = jnp.dot(hid, wf2_ref[...], preferred_element_type=_F32) + _row(bf2_ref)
    o_ref[...] = _ln(x1 + ff, _row(g2g_ref), _row(g2b_ref))


def _tc_global(xg, qg, kg, vg, wgoT, bgo, g1g, g1b, wf1T, bf1, wf2T, bf2,
               g2g, g2b):
    grid = GPAD // QB
    qblk = pl.BlockSpec((QB, C), lambda i: (i, 0))
    kfull = pl.BlockSpec((GPAD, C), lambda i: (0, 0))
    full = pl.BlockSpec((C, C), lambda i: (0, 0))
    brow = pl.BlockSpec((8, C), lambda i: (0, 0))
    w1s = pl.BlockSpec((C, 4 * C), lambda i: (0, 0))
    b1s = pl.BlockSpec((8, 4 * C), lambda i: (0, 0))
    w2s = pl.BlockSpec((4 * C, C), lambda i: (0, 0))
    return pl.pallas_call(
        _global_body,
        grid=(grid,),
        in_specs=[qblk, qblk, kfull, kfull, full, brow, brow, brow,
                  w1s, b1s, w2s, brow, brow, brow],
        out_specs=qblk,
        out_shape=jax.ShapeDtypeStruct((GPAD, C), _F32),
    )(xg, qg, kg, vg, wgoT, bgo, g1g, g1b, wf1T, bf1, wf2T, bf2, g2g, g2b)


NB_BLK = 1024


def _nearest_body(pos_ref, pgT_ref, o_ref):
    pos = pos_ref[...]
    pgT = pgT_ref[...]
    colid = lax.broadcasted_iota(jnp.int32, (NB_BLK, GPAD), 1)
    d2 = jnp.zeros((NB_BLK, GPAD), _F32)
    for cdim in range(3):
        dc = pos[:, cdim:cdim + 1] - pgT[cdim:cdim + 1, :]
        d2 = d2 + dc * dc
    d = jnp.sqrt(d2)
    d = jnp.where(colid < NG, d, jnp.inf)
    dmin = jnp.min(d, axis=-1, keepdims=True)
    near = jnp.min(jnp.where(d == dmin, colid, jnp.int32(2**30)), axis=-1)
    o_ref[...] = near.astype(jnp.int32).reshape(NB_BLK // C, C)


def _tc_nearest(pos16, posgT):
    grid = NPAD // NB_BLK
    return pl.pallas_call(
        _nearest_body,
        grid=(grid,),
        in_specs=[pl.BlockSpec((NB_BLK, C), lambda i: (i, 0)),
                  pl.BlockSpec((C, GPAD), lambda i: (0, 0))],
        out_specs=pl.BlockSpec((NB_BLK // C, C), lambda i: (i, 0)),
        out_shape=jax.ShapeDtypeStruct((NPAD // C, C), jnp.int32),
    )(pos16, posgT)


def _fuse_body(xl_ref, xgf_ref, wa_ref, wb_ref, bf_ref, g_ref, b_ref, o_ref):
    f = (jnp.dot(xl_ref[...], wa_ref[...], preferred_element_type=_F32)
         + jnp.dot(xgf_ref[...], wb_ref[...], preferred_element_type=_F32)
         + _row(bf_ref))
    f = _ln(f, _row(g_ref), _row(b_ref))
    o_ref[...] = _gelu(f)


def _tc_fuse(xl, xgf, waT, wbT, bf, g, b):
    blk = 1024
    grid = NPAD // blk
    rblk = pl.BlockSpec((blk, C), lambda i: (i, 0))
    full = pl.BlockSpec((C, C), lambda i: (0, 0))
    brow = pl.BlockSpec((8, C), lambda i: (0, 0))
    return pl.pallas_call(
        _fuse_body,
        grid=(grid,),
        in_specs=[rblk, rblk, full, full, brow, brow, brow],
        out_specs=rblk,
        out_shape=jax.ShapeDtypeStruct((NPAD, C), _F32),
    )(xl, xgf, waT, wbT, bf, g, b)


def _tile8(v):
    return jnp.tile(v[None, :], (8, 1))


def kernel(x, pos, idx, params):
    p = params
    xp = jnp.zeros((NPAD, C), _F32).at[:N].set(x)
    pos16 = jnp.zeros((NPAD, C), _F32).at[:N, :3].set(pos)
    idxp = jnp.zeros((NPAD, K), jnp.int32).at[:N].set(idx)
    idxflat = idxp.reshape(BF)

    wqT, bq = p['q'][0].T, _tile8(p['q'][1])
    wkT, bk = p['k'][0].T, _tile8(p['k'][1])
    wvT, bv = p['v'][0].T, _tile8(p['v'][1])
    woT, bo = p['o'][0].T, _tile8(p['o'][1])
    w1p = jnp.zeros((16, 64), _F32).at[:3].set(p['pe1'][0].T)
    bd1 = jnp.kron(jnp.eye(8, dtype=_F32), w1p)
    b1 = jnp.tile(p['pe1'][1], 8)[None, :].repeat(8, 0)
    bd2 = jnp.kron(jnp.eye(8, dtype=_F32), p['pe2'][0].T)
    b2 = jnp.tile(p['pe2'][1], 8)[None, :].repeat(8, 0)
    lng, lnb = _tile8(p['ln_local'][0]), _tile8(p['ln_local'][1])
    wqkvT, bqkv = p['qkv'][0].T, _tile8(p['qkv'][1])
    wgoT, bgo = p['go'][0].T, _tile8(p['go'][1])
    g1g, g1b = _tile8(p['gn1'][0]), _tile8(p['gn1'][1])
    g2g, g2b = _tile8(p['gn2'][0]), _tile8(p['gn2'][1])
    wf1T, bf1 = p['ffn1'][0].T, _tile8(p['ffn1'][1])
    wf2T, bf2 = p['ffn2'][0].T, _tile8(p['ffn2'][1])
    waT = p['fuse'][0][:, :C].T
    wbT = p['fuse'][0][:, C:].T
    bfu = _tile8(p['fuse'][1])
    flg, flb = _tile8(p['fuse_ln'][0]), _tile8(p['fuse_ln'][1])

    gidx = jnp.zeros((GPAD,), jnp.int32).at[:NG].set(
        jnp.asarray(_get_gidx(), jnp.int32))

    px = jnp.zeros((NPAD,), _F32).at[:N].set(pos[:, 0])
    py = jnp.zeros((NPAD,), _F32).at[:N].set(pos[:, 1])
    pz = jnp.zeros((NPAD,), _F32).at[:N].set(pos[:, 2])

    xls = []
    for h in range(2):
        idx_h = lax.slice_in_dim(idxflat, h * BFH, (h + 1) * BFH)
        xf, relflat = _sc_gather_half(h * NHALF, idx_h, xp, px, py, pz)
        relp = relflat.reshape(BFH * 16 // C, C)
        xp_h = lax.slice_in_dim(xp, h * NHALF, (h + 1) * NHALF)
        xls.append(_tc_local_attn(xp_h, xf, relp, wqT, bq, wkT, bk, wvT, bv,
                                  bd1, b1, bd2, b2, woT, bo, lng, lnb))
    x_local = jnp.concatenate(xls, axis=0)

    xg_in, posg = _sc_gather_anchors(gidx, x_local, pos16)
    qkv = _tc_qkv(xg_in, wqkvT, bqkv)
    qg, kg, vg = qkv[:, :C], qkv[:, C:2 * C], qkv[:, 2 * C:]
    xg = _tc_global(xg_in, qg, kg, vg, wgoT, bgo, g1g, g1b,
                    wf1T, bf1, wf2T, bf2, g2g, g2b)

    nearest = _tc_nearest(pos16, posg.T).reshape(NPAD)
    xgf = _sc_gather_rows(nearest, xg)

    out = _tc_fuse(x_local, xgf, waT, wbT, bfu, flg, flb)
    return out[:N]
